# initial kernel scaffold (unmeasured)
import jax
import jax.numpy as jnp
from jax import lax
from jax.experimental import pallas as pl
from jax.experimental.pallas import tpu as pltpu


def kernel(
    x,
):
    def body(*refs):
        pass

    out_shape = jax.ShapeDtypeStruct(..., jnp.float32)
    return pl.pallas_call(body, out_shape=out_shape)(...)


# baseline (device time: 80030 ns/iter reference)
import jax
import jax.numpy as jnp
from jax import lax
from jax.experimental import pallas as pl
from jax.experimental.pallas import tpu as pltpu

N_Z = 4


def kernel(x):
    m, n = x.shape

    def body(x_ref, out_ref, comm_ref, send_sems, recv_sems):
        my_x = lax.axis_index("x")
        my_y = lax.axis_index("y")
        my_z = lax.axis_index("z")
        right = (my_z + 1) % N_Z
        left = (my_z - 1) % N_Z

        barrier_sem = pltpu.get_barrier_semaphore()
        for nbr in (left, right):
            pl.semaphore_signal(
                barrier_sem,
                inc=1,
                device_id=(my_x, my_y, nbr),
                device_id_type=pl.DeviceIdType.MESH,
            )
        pl.semaphore_wait(barrier_sem, 2)

        out_ref[:, :] = x_ref[:, :]
        comm_ref[0, :, :] = x_ref[:, :]

        for h in range(N_Z - 1):
            rdma = pltpu.make_async_remote_copy(
                src_ref=comm_ref.at[h],
                dst_ref=comm_ref.at[h + 1],
                send_sem=send_sems.at[h],
                recv_sem=recv_sems.at[h],
                device_id=(my_x, my_y, right),
                device_id_type=pl.DeviceIdType.MESH,
            )
            rdma.start()
            rdma.wait()
            out_ref[:, :] = out_ref[:, :] + comm_ref[h + 1, :, :]

    return pl.pallas_call(
        body,
        out_shape=jax.ShapeDtypeStruct((m, n), x.dtype),
        in_specs=[pl.BlockSpec(memory_space=pltpu.VMEM)],
        out_specs=pl.BlockSpec(memory_space=pltpu.VMEM),
        scratch_shapes=[
            pltpu.VMEM((N_Z, m, n), x.dtype),
            pltpu.SemaphoreType.DMA((N_Z - 1,)),
            pltpu.SemaphoreType.DMA((N_Z - 1,)),
        ],
        compiler_params=pltpu.CompilerParams(collective_id=0),
    )(x)


# device time: 51608 ns/iter; 1.5507x vs baseline; 1.5507x over previous
import jax
import jax.numpy as jnp
from jax import lax
from jax.experimental import pallas as pl
from jax.experimental.pallas import tpu as pltpu

N_Z = 4
N_STEPS = 2 * (N_Z - 1)


def kernel(x):
    m, n = x.shape
    half = m // 2
    rows = half // N_Z

    def body(x_ref, out_ref, recv_buf, send_sems, recv_sems):
        my_x = lax.axis_index("x")
        my_y = lax.axis_index("y")
        my_z = lax.axis_index("z")
        right = (my_z + 1) % N_Z
        left = (my_z - 1) % N_Z

        barrier_sem = pltpu.get_barrier_semaphore()
        for nbr in (left, right):
            pl.semaphore_signal(
                barrier_sem,
                inc=1,
                device_id=(my_x, my_y, nbr),
                device_id_type=pl.DeviceIdType.MESH,
            )
        pl.semaphore_wait(barrier_sem, 2)

        out_ref[:, :] = x_ref[:, :]

        def row_off(d, idx):
            return d * half + idx * rows

        for s in range(N_STEPS):
            rdmas = []
            for d, dst in ((0, right), (1, left)):
                sgn = 1 - 2 * d
                send_idx = (my_z - sgn * s) % N_Z
                rdma = pltpu.make_async_remote_copy(
                    src_ref=out_ref.at[pl.ds(row_off(d, send_idx), rows), :],
                    dst_ref=recv_buf.at[d, s],
                    send_sem=send_sems.at[d, s],
                    recv_sem=recv_sems.at[d, s],
                    device_id=(my_x, my_y, dst),
                    device_id_type=pl.DeviceIdType.MESH,
                )
                rdma.start()
                rdmas.append(rdma)
            for d, rdma in enumerate(rdmas):
                sgn = 1 - 2 * d
                recv_idx = (my_z - sgn * (s + 1)) % N_Z
                rdma.wait()
                off = row_off(d, recv_idx)
                if s < N_Z - 1:
                    out_ref[pl.ds(off, rows), :] = (
                        out_ref[pl.ds(off, rows), :] + recv_buf[d, s, :, :]
                    )
                else:
                    out_ref[pl.ds(off, rows), :] = recv_buf[d, s, :, :]

    return pl.pallas_call(
        body,
        out_shape=jax.ShapeDtypeStruct((m, n), x.dtype),
        in_specs=[pl.BlockSpec(memory_space=pltpu.VMEM)],
        out_specs=pl.BlockSpec(memory_space=pltpu.VMEM),
        scratch_shapes=[
            pltpu.VMEM((2, N_STEPS, rows, n), x.dtype),
            pltpu.SemaphoreType.DMA((2, N_STEPS)),
            pltpu.SemaphoreType.DMA((2, N_STEPS)),
        ],
        compiler_params=pltpu.CompilerParams(collective_id=0),
    )(x)
